# baseline (device time: 29273 ns/iter reference)
import jax
import jax.numpy as jnp
from jax import lax
from jax.experimental import pallas as pl
from jax.experimental.pallas import tpu as pltpu

N_DEV = 4
B, SQ, D_MODEL = 2, 256, 512
SKV = 1024
HQ, DH = 16, 64
H_LOC = HQ // N_DEV
SKV_LOC = SKV // N_DEV
SQ_C = SQ // N_DEV
BF16 = jnp.bfloat16
INT8 = jnp.int8
QSCALE = 127.0 / 4.5
PSCALE = 127.0 / 0.12


def kernel(x, Wq, K_ext, V_ext, Wo):
    kt = jnp.transpose(K_ext, (0, 2, 3, 1))
    vt = jnp.transpose(V_ext, (0, 2, 3, 1))

    def body(x_ref, wq_ref, k_ref, v_ref, wo_ref, out_ref,
             xv, wqv, wov, ktv, vtv,
             kstage, kvsend, kvrecv, pchunks, rsbuf, agbuf,
             cp_sems, kv_send_sems, kv_recv_sems,
             rs_send_sems, rs_recv_sems, ag_send_sems, ag_recv_sems):
        my = lax.axis_index("i")

        cps = []
        for i, (s, t) in enumerate([(k_ref, ktv), (v_ref, vtv), (x_ref, xv),
                                    (wq_ref, wqv), (wo_ref, wov)]):
            c = pltpu.make_async_copy(s, t, cp_sems.at[i])
            c.start()
            cps.append(c)
        cps[0].wait()
        cps[1].wait()

        kstage[:, 0] = jnp.transpose(ktv[...], (1, 0, 2, 3)).astype(BF16)
        kstage[:, 1] = jnp.transpose(vtv[...], (1, 0, 2, 3)).astype(BF16)

        barrier = pltpu.get_barrier_semaphore()
        for d in range(1, N_DEV):
            peer = lax.rem(my + d, N_DEV)
            pl.semaphore_signal(barrier, inc=1, device_id=(peer,),
                                device_id_type=pl.DeviceIdType.MESH)
        pl.semaphore_wait(barrier, N_DEV - 1)

        kv_rdmas = []
        for d in (1, 3, 2):
            j = lax.rem(my + d, N_DEV)
            grp = kstage[pl.ds(j * H_LOC, H_LOC)]
            kvsend[pl.ds(j * H_LOC, H_LOC)] = jnp.clip(
                jnp.round(grp.astype(jnp.float32) * QSCALE), -127, 127
            ).astype(INT8)
            r = pltpu.make_async_remote_copy(
                src_ref=kvsend.at[pl.ds(j * H_LOC, H_LOC)],
                dst_ref=kvrecv.at[my],
                send_sem=kv_send_sems.at[j],
                recv_sem=kv_recv_sems.at[my],
                device_id=(j,),
                device_id_type=pl.DeviceIdType.MESH,
            )
            r.start()
            kv_rdmas.append(r)

        cps[2].wait()
        cps[3].wait()
        wq = wqv[...].astype(BF16)
        qcat = lax.dot(
            jnp.concatenate([xv[b] for b in range(B)], axis=0).astype(BF16),
            wq, preferred_element_type=jnp.float32).astype(BF16)
        qs = [qcat[b * SQ:(b + 1) * SQ] for b in range(B)]

        qb = lax.broadcasted_iota(jnp.int32, (SQ, SKV_LOC), 0) // SQ_C
        kb = lax.broadcasted_iota(jnp.int32, (SQ, SKV_LOC), 1) // SQ_C
        mask2 = qb == kb

        esum = [[None] * H_LOC for _ in range(B)]
        ctxa = [[None] * H_LOC for _ in range(B)]

        def accum(chunk):
            chunk = chunk.astype(BF16)
            for b in range(B):
                for h in range(H_LOC):
                    q_bh = qs[b][:, h * DH:(h + 1) * DH]
                    s_ = lax.dot(
                        q_bh, chunk[h, 0, b],
                        preferred_element_type=jnp.float32) * (0.125 / QSCALE)
                    e_ = jnp.where(mask2, jnp.exp(s_), 0.0)
                    row = jnp.sum(e_, axis=1, keepdims=True)
                    pv = lax.dot_general(
                        e_.astype(BF16), chunk[h, 1, b], (((1,), (1,)), ((), ())),
                        preferred_element_type=jnp.float32)
                    esum[b][h] = row if esum[b][h] is None else esum[b][h] + row
                    ctxa[b][h] = pv if ctxa[b][h] is None else ctxa[b][h] + pv

        accum(kstage[pl.ds(my * H_LOC, H_LOC)] * QSCALE)
        for d in (1, 3, 2):
            j = lax.rem(my + d, N_DEV)
            pltpu.make_async_remote_copy(
                src_ref=kvsend.at[pl.ds(0, H_LOC)],
                dst_ref=kvrecv.at[j],
                send_sem=kv_send_sems.at[j],
                recv_sem=kv_recv_sems.at[j],
                device_id=(j,),
                device_id_type=pl.DeviceIdType.MESH,
            ).wait_recv()
            accum(kvrecv[pl.ds(j, 1)][0])

        cps[4].wait()
        wo16 = wov[...].astype(BF16)
        for c in range(N_DEV):
            rows = []
            for b in range(B):
                cols = []
                for h in range(H_LOC):
                    num = ctxa[b][h][c * SQ_C:(c + 1) * SQ_C]
                    den = esum[b][h][c * SQ_C:(c + 1) * SQ_C] * QSCALE
                    cols.append((num / den).astype(BF16))
                rows.append(jnp.concatenate(cols, axis=1))
            pchunks[c] = jnp.clip(jnp.round(lax.dot(
                jnp.concatenate(rows, axis=0), wo16,
                preferred_element_type=jnp.float32
            ) * PSCALE), -127, 127).astype(INT8).reshape(B, SQ_C, D_MODEL)

            @pl.when(my != c)
            def _():
                pltpu.make_async_remote_copy(
                    src_ref=pchunks.at[c],
                    dst_ref=rsbuf.at[my],
                    send_sem=rs_send_sems.at[c],
                    recv_sem=rs_recv_sems.at[my],
                    device_id=(c,),
                    device_id_type=pl.DeviceIdType.MESH,
                ).start()

        rsbuf[pl.ds(my, 1)] = pchunks[pl.ds(my, 1)]
        for d in range(1, N_DEV):
            j = lax.rem(my + d, N_DEV)
            pltpu.make_async_remote_copy(
                src_ref=pchunks.at[j],
                dst_ref=rsbuf.at[j],
                send_sem=rs_send_sems.at[j],
                recv_sem=rs_recv_sems.at[j],
                device_id=(j,),
                device_id_type=pl.DeviceIdType.MESH,
            ).wait_recv()

        my_sum = jnp.sum(rsbuf[...].astype(jnp.float32), axis=0) * (1.0 / PSCALE)
        agbuf[...] = my_sum.astype(BF16)
        out_ref[:, pl.ds(my * SQ_C, SQ_C)] = agbuf[...]

        ag_rdmas = []
        for d in range(1, N_DEV):
            j = lax.rem(my + d, N_DEV)
            r = pltpu.make_async_remote_copy(
                src_ref=agbuf,
                dst_ref=out_ref.at[:, pl.ds(my * SQ_C, SQ_C)],
                send_sem=ag_send_sems.at[j],
                recv_sem=ag_recv_sems.at[my],
                device_id=(j,),
                device_id_type=pl.DeviceIdType.MESH,
            )
            r.start()
            ag_rdmas.append(r)
        for d in range(1, N_DEV):
            j = lax.rem(my + d, N_DEV)
            pltpu.make_async_remote_copy(
                src_ref=agbuf,
                dst_ref=out_ref.at[:, pl.ds(j * SQ_C, SQ_C)],
                send_sem=ag_send_sems.at[j],
                recv_sem=ag_recv_sems.at[j],
                device_id=(j,),
                device_id_type=pl.DeviceIdType.MESH,
            ).wait_recv()

        for r in kv_rdmas + ag_rdmas:
            r.wait_send()
        for c in range(N_DEV):
            @pl.when(my != c)
            def _():
                pltpu.make_async_remote_copy(
                    src_ref=pchunks.at[c],
                    dst_ref=rsbuf.at[my],
                    send_sem=rs_send_sems.at[c],
                    recv_sem=rs_recv_sems.at[my],
                    device_id=(c,),
                    device_id_type=pl.DeviceIdType.MESH,
                ).wait_send()

    return pl.pallas_call(
        body,
        out_shape=jax.ShapeDtypeStruct((B, SQ, D_MODEL), BF16),
        in_specs=[pl.BlockSpec(memory_space=pl.ANY)] * 5,
        out_specs=pl.BlockSpec(memory_space=pltpu.VMEM),
        scratch_shapes=[
            pltpu.VMEM((B, SQ, D_MODEL), jnp.float32),
            pltpu.VMEM((D_MODEL, HQ * DH // N_DEV), jnp.float32),
            pltpu.VMEM((HQ * DH // N_DEV, D_MODEL), jnp.float32),
            pltpu.VMEM((B, HQ, DH, SKV_LOC), jnp.float32),
            pltpu.VMEM((B, HQ, DH, SKV_LOC), jnp.float32),
            pltpu.VMEM((HQ, 2, B, DH, SKV_LOC), BF16),
            pltpu.VMEM((HQ, 2, B, DH, SKV_LOC), INT8),
            pltpu.VMEM((N_DEV, H_LOC, 2, B, DH, SKV_LOC), INT8),
            pltpu.VMEM((N_DEV, B, SQ_C, D_MODEL), INT8),
            pltpu.VMEM((N_DEV, B, SQ_C, D_MODEL), INT8),
            pltpu.VMEM((B, SQ_C, D_MODEL), BF16),
            pltpu.SemaphoreType.DMA((5,)),
            pltpu.SemaphoreType.DMA((N_DEV,)),
            pltpu.SemaphoreType.DMA((N_DEV,)),
            pltpu.SemaphoreType.DMA((N_DEV,)),
            pltpu.SemaphoreType.DMA((N_DEV,)),
            pltpu.SemaphoreType.DMA((N_DEV,)),
            pltpu.SemaphoreType.DMA((N_DEV,)),
        ],
        compiler_params=pltpu.CompilerParams(collective_id=0),
    )(x, Wq, kt, vt, Wo)


# device time: 28812 ns/iter; 1.0160x vs baseline; 1.0160x over previous
import jax
import jax.numpy as jnp
from jax import lax
from jax.experimental import pallas as pl
from jax.experimental.pallas import tpu as pltpu

N_DEV = 4
B, SQ, D_MODEL = 2, 256, 512
SKV = 1024
HQ, DH = 16, 64
H_LOC = HQ // N_DEV
SKV_LOC = SKV // N_DEV
SQ_C = SQ // N_DEV
BF16 = jnp.bfloat16
INT8 = jnp.int8
QSCALE = 127.0 / 4.5
PSCALE = 127.0 / 0.12


def kernel(x, Wq, K_ext, V_ext, Wo):
    kt = jnp.transpose(K_ext, (0, 2, 3, 1))
    vt = jnp.transpose(V_ext, (0, 2, 3, 1))

    def body(x_ref, wq_ref, k_ref, v_ref, wo_ref, out_ref,
             xv, wqv, wov,
             kstage, kvsend, kvrecv, pchunks, rsbuf, agbuf,
             cp_sems, kv_send_sems, kv_recv_sems,
             rs_send_sems, rs_recv_sems, ag_send_sems, ag_recv_sems):
        my = lax.axis_index("i")

        cps = []
        for i, (s, t) in enumerate([(x_ref, xv), (wq_ref, wqv), (wo_ref, wov)]):
            c = pltpu.make_async_copy(s, t, cp_sems.at[i])
            c.start()
            cps.append(c)

        kstage[:, 0] = jnp.transpose(k_ref[...], (1, 0, 2, 3)).astype(BF16)
        kstage[:, 1] = jnp.transpose(v_ref[...], (1, 0, 2, 3)).astype(BF16)

        barrier = pltpu.get_barrier_semaphore()
        for d in range(1, N_DEV):
            peer = lax.rem(my + d, N_DEV)
            pl.semaphore_signal(barrier, inc=1, device_id=(peer,),
                                device_id_type=pl.DeviceIdType.MESH)
        pl.semaphore_wait(barrier, N_DEV - 1)

        kv_rdmas = []
        for d in (1, 3, 2):
            j = lax.rem(my + d, N_DEV)
            grp = kstage[pl.ds(j * H_LOC, H_LOC)]
            kvsend[pl.ds(j * H_LOC, H_LOC)] = jnp.clip(
                jnp.round(grp.astype(jnp.float32) * QSCALE), -127, 127
            ).astype(INT8)
            r = pltpu.make_async_remote_copy(
                src_ref=kvsend.at[pl.ds(j * H_LOC, H_LOC)],
                dst_ref=kvrecv.at[my],
                send_sem=kv_send_sems.at[j],
                recv_sem=kv_recv_sems.at[my],
                device_id=(j,),
                device_id_type=pl.DeviceIdType.MESH,
            )
            r.start()
            kv_rdmas.append(r)

        cps[0].wait()
        cps[1].wait()
        wq = wqv[...].astype(BF16)
        qcat = lax.dot(
            jnp.concatenate([xv[b] for b in range(B)], axis=0).astype(BF16),
            wq, preferred_element_type=jnp.float32).astype(BF16)
        qs = [qcat[b * SQ:(b + 1) * SQ] for b in range(B)]

        qb = lax.broadcasted_iota(jnp.int32, (SQ, SKV_LOC), 0) // SQ_C
        kb = lax.broadcasted_iota(jnp.int32, (SQ, SKV_LOC), 1) // SQ_C
        mask2 = qb == kb

        esum = [[None] * H_LOC for _ in range(B)]
        ctxa = [[None] * H_LOC for _ in range(B)]

        def accum(chunk):
            chunk = chunk.astype(BF16)
            for b in range(B):
                for h in range(H_LOC):
                    q_bh = qs[b][:, h * DH:(h + 1) * DH]
                    s_ = lax.dot(
                        q_bh, chunk[h, 0, b],
                        preferred_element_type=jnp.float32) * (0.125 / QSCALE)
                    e_ = jnp.where(mask2, jnp.exp(s_), 0.0)
                    row = jnp.sum(e_, axis=1, keepdims=True)
                    pv = lax.dot_general(
                        e_.astype(BF16), chunk[h, 1, b], (((1,), (1,)), ((), ())),
                        preferred_element_type=jnp.float32)
                    esum[b][h] = row if esum[b][h] is None else esum[b][h] + row
                    ctxa[b][h] = pv if ctxa[b][h] is None else ctxa[b][h] + pv

        accum(kstage[pl.ds(my * H_LOC, H_LOC)] * QSCALE)
        for d in (1, 3, 2):
            j = lax.rem(my + d, N_DEV)
            pltpu.make_async_remote_copy(
                src_ref=kvsend.at[pl.ds(0, H_LOC)],
                dst_ref=kvrecv.at[j],
                send_sem=kv_send_sems.at[j],
                recv_sem=kv_recv_sems.at[j],
                device_id=(j,),
                device_id_type=pl.DeviceIdType.MESH,
            ).wait_recv()
            accum(kvrecv[pl.ds(j, 1)][0])

        cps[2].wait()
        wo16 = wov[...].astype(BF16)
        for c in range(N_DEV):
            rows = []
            for b in range(B):
                cols = []
                for h in range(H_LOC):
                    num = ctxa[b][h][c * SQ_C:(c + 1) * SQ_C]
                    den = esum[b][h][c * SQ_C:(c + 1) * SQ_C] * QSCALE
                    cols.append((num / den).astype(BF16))
                rows.append(jnp.concatenate(cols, axis=1))
            pchunks[c] = jnp.clip(jnp.round(lax.dot(
                jnp.concatenate(rows, axis=0), wo16,
                preferred_element_type=jnp.float32
            ) * PSCALE), -127, 127).astype(INT8).reshape(B, SQ_C, D_MODEL)

            @pl.when(my != c)
            def _():
                pltpu.make_async_remote_copy(
                    src_ref=pchunks.at[c],
                    dst_ref=rsbuf.at[my],
                    send_sem=rs_send_sems.at[c],
                    recv_sem=rs_recv_sems.at[my],
                    device_id=(c,),
                    device_id_type=pl.DeviceIdType.MESH,
                ).start()

        rsbuf[pl.ds(my, 1)] = pchunks[pl.ds(my, 1)]
        for d in range(1, N_DEV):
            j = lax.rem(my + d, N_DEV)
            pltpu.make_async_remote_copy(
                src_ref=pchunks.at[j],
                dst_ref=rsbuf.at[j],
                send_sem=rs_send_sems.at[j],
                recv_sem=rs_recv_sems.at[j],
                device_id=(j,),
                device_id_type=pl.DeviceIdType.MESH,
            ).wait_recv()

        my_sum = jnp.sum(rsbuf[...].astype(jnp.float32), axis=0) * (1.0 / PSCALE)
        agbuf[...] = my_sum.astype(BF16)
        out_ref[:, pl.ds(my * SQ_C, SQ_C)] = agbuf[...]

        ag_rdmas = []
        for d in range(1, N_DEV):
            j = lax.rem(my + d, N_DEV)
            r = pltpu.make_async_remote_copy(
                src_ref=agbuf,
                dst_ref=out_ref.at[:, pl.ds(my * SQ_C, SQ_C)],
                send_sem=ag_send_sems.at[j],
                recv_sem=ag_recv_sems.at[my],
                device_id=(j,),
                device_id_type=pl.DeviceIdType.MESH,
            )
            r.start()
            ag_rdmas.append(r)
        for d in range(1, N_DEV):
            j = lax.rem(my + d, N_DEV)
            pltpu.make_async_remote_copy(
                src_ref=agbuf,
                dst_ref=out_ref.at[:, pl.ds(j * SQ_C, SQ_C)],
                send_sem=ag_send_sems.at[j],
                recv_sem=ag_recv_sems.at[j],
                device_id=(j,),
                device_id_type=pl.DeviceIdType.MESH,
            ).wait_recv()

        for r in kv_rdmas + ag_rdmas:
            r.wait_send()
        for c in range(N_DEV):
            @pl.when(my != c)
            def _():
                pltpu.make_async_remote_copy(
                    src_ref=pchunks.at[c],
                    dst_ref=rsbuf.at[my],
                    send_sem=rs_send_sems.at[c],
                    recv_sem=rs_recv_sems.at[my],
                    device_id=(c,),
                    device_id_type=pl.DeviceIdType.MESH,
                ).wait_send()

    return pl.pallas_call(
        body,
        out_shape=jax.ShapeDtypeStruct((B, SQ, D_MODEL), BF16),
        in_specs=[
            pl.BlockSpec(memory_space=pl.ANY),
            pl.BlockSpec(memory_space=pl.ANY),
            pl.BlockSpec(memory_space=pltpu.VMEM),
            pl.BlockSpec(memory_space=pltpu.VMEM),
            pl.BlockSpec(memory_space=pl.ANY),
        ],
        out_specs=pl.BlockSpec(memory_space=pltpu.VMEM),
        scratch_shapes=[
            pltpu.VMEM((B, SQ, D_MODEL), jnp.float32),
            pltpu.VMEM((D_MODEL, HQ * DH // N_DEV), jnp.float32),
            pltpu.VMEM((HQ * DH // N_DEV, D_MODEL), jnp.float32),
            pltpu.VMEM((HQ, 2, B, DH, SKV_LOC), BF16),
            pltpu.VMEM((HQ, 2, B, DH, SKV_LOC), INT8),
            pltpu.VMEM((N_DEV, H_LOC, 2, B, DH, SKV_LOC), INT8),
            pltpu.VMEM((N_DEV, B, SQ_C, D_MODEL), INT8),
            pltpu.VMEM((N_DEV, B, SQ_C, D_MODEL), INT8),
            pltpu.VMEM((B, SQ_C, D_MODEL), BF16),
            pltpu.SemaphoreType.DMA((3,)),
            pltpu.SemaphoreType.DMA((N_DEV,)),
            pltpu.SemaphoreType.DMA((N_DEV,)),
            pltpu.SemaphoreType.DMA((N_DEV,)),
            pltpu.SemaphoreType.DMA((N_DEV,)),
            pltpu.SemaphoreType.DMA((N_DEV,)),
            pltpu.SemaphoreType.DMA((N_DEV,)),
        ],
        compiler_params=pltpu.CompilerParams(collective_id=0),
    )(x, Wq, kt, vt, Wo)
